# Initial kernel scaffold; baseline (speedup 1.0000x reference)
import jax
import jax.numpy as jnp
from jax.experimental import pallas as pl


def kernel(x, edge_index, dropout, params):
    raise NotImplementedError("write your pallas kernel here")


# speedup vs baseline: 5.7258x; 5.7258x over previous
import functools

import jax
import jax.numpy as jnp
from jax import lax
from jax.experimental import pallas as pl
from jax.experimental.pallas import tpu as pltpu
from jax.experimental.pallas import tpu_sc as plsc

_N = 50000
_E = 800000
_F_IN = 61
_H = 32
_HH = 16

_NW = 32
_K = 128
_NCHUNK = 196
_E_PAD = _NW * _K * _NCHUNK
_N_PAD = 50176
_RPT = _N_PAD // 16
_ZR = 224

_BLK = 5000


_sc_mesh = plsc.VectorSubcoreMesh(core_axis_name="c", subcore_axis_name="s")


@functools.partial(
    pl.kernel,
    mesh=_sc_mesh,
    out_type=jax.ShapeDtypeStruct((2, 2, _N_PAD, _HH), jnp.float32),
    scratch_types=[
        pltpu.VMEM((_NCHUNK, _K), jnp.int32),
        pltpu.VMEM((_NCHUNK, _K), jnp.int32),
        pltpu.VMEM((_K, _HH), jnp.float32),
        pltpu.VMEM((_ZR, _HH), jnp.float32),
        pltpu.VMEM((_ZR, _HH), jnp.float32),
        pltpu.VMEM_SHARED((_N_PAD, _HH), jnp.float32),
        pltpu.SemaphoreType.DMA,
    ],
    compiler_params=pltpu.CompilerParams(use_tc_tiling_on_sc=False),
)
def _seg_sum_sc(pa_hbm, pb_hbm, src_hbm, dst_hbm, out_hbm,
                src_v, dst_v, rows_v, zbuf, obuf, agg, sem):
    cid = lax.axis_index("c")
    sid = lax.axis_index("s")
    wid = sid * 2 + cid

    pltpu.sync_copy(src_hbm.at[wid], src_v)
    pltpu.sync_copy(dst_hbm.at[wid], dst_v)

    z16 = jnp.zeros((16,), jnp.float32)

    def _zrow(i, carry):
        zbuf[i, pl.ds(0, _HH)] = z16
        return carry

    lax.fori_loop(0, _ZR, _zrow, 0)

    row0 = sid * _RPT

    for ph, p_hbm in enumerate((pa_hbm, pb_hbm)):
        def _zcp(k, carry):
            pltpu.sync_copy(zbuf, agg.at[pl.ds(row0 + k * _ZR, _ZR)])
            return carry

        lax.fori_loop(0, _RPT // _ZR, _zcp, 0)
        plsc.subcore_barrier()

        def _edge_chunk(j, carry):
            pltpu.async_copy(p_hbm.at[src_v.at[j]], rows_v, sem).wait()
            pltpu.sync_copy(rows_v, agg.at[dst_v.at[j]], add=True)
            return carry

        lax.fori_loop(0, _NCHUNK, _edge_chunk, 0)
        plsc.subcore_barrier()

        def _ocp(k, carry):
            sl = pl.ds(row0 + k * _ZR, _ZR)
            pltpu.sync_copy(agg.at[sl], obuf)
            pltpu.sync_copy(obuf, out_hbm.at[ph, cid, sl])
            return carry

        lax.fori_loop(0, _RPT // _ZR, _ocp, 0)
        plsc.subcore_barrier()


def _split_store(res, oa_ref, ob_ref):
    oa_ref[...] = res[:, :_HH]
    ob_ref[...] = res[:, _HH:]


def _init_body(x_ref, w_ref, oa_ref, ob_ref):
    res = jnp.dot(x_ref[...], w_ref[...], preferred_element_type=jnp.float32)
    _split_store(res, oa_ref, ob_ref)


def _gin_epilogue(pa_ref, pb_ref, a_ref, w2_ref, v_ref):
    qa = pa_ref[...] + a_ref[0, 0] + a_ref[0, 1] + v_ref[0:1, :_HH]
    qb = pb_ref[...] + a_ref[1, 0] + a_ref[1, 1] + v_ref[0:1, _HH:]
    r = jnp.maximum(jnp.concatenate([qa, qb], axis=1), 0.0)
    z = jnp.dot(r, w2_ref[...], preferred_element_type=jnp.float32)
    u = jnp.maximum(z + v_ref[1:2, :], 0.0)
    return u * v_ref[2:3, :] + v_ref[3:4, :]


def _stage_body(pa_ref, pb_ref, a_ref, w2_ref, w1n_ref, v_ref,
                oa_ref, ob_ref):
    h = _gin_epilogue(pa_ref, pb_ref, a_ref, w2_ref, v_ref)
    res = jnp.dot(h, w1n_ref[...], preferred_element_type=jnp.float32)
    _split_store(res, oa_ref, ob_ref)


def _final_body(pa_ref, pb_ref, a_ref, w2_ref, v_ref, f1w_ref, f1b_ref,
                f2w_ref, f2b_ref, o_ref):
    h = _gin_epilogue(pa_ref, pb_ref, a_ref, w2_ref, v_ref)
    t = jnp.maximum(
        jnp.dot(h, f1w_ref[...], preferred_element_type=jnp.float32)
        + f1b_ref[0:1, :], 0.0)
    o = jnp.dot(t, f2w_ref[...], preferred_element_type=jnp.float32) \
        + f2b_ref[0:1, :]
    m = jnp.max(o, axis=1, keepdims=True)
    lse = m + jnp.log(jnp.sum(jnp.exp(o - m), axis=1, keepdims=True))
    o_ref[...] = o - lse


def _row_spec(cols):
    return pl.BlockSpec((_BLK, cols), lambda b: (b, 0))


def _full_spec(shape):
    nd = len(shape)
    return pl.BlockSpec(shape, lambda b: (0,) * nd)


_agg_spec = pl.BlockSpec((2, 2, _BLK, _HH), lambda b: (0, 0, b, 0))
_grid = (_N // _BLK,)
_half_shapes = (jax.ShapeDtypeStruct((_N, _HH), jnp.float32),
                jax.ShapeDtypeStruct((_N, _HH), jnp.float32))


def _tc_init(x_pad, w1p):
    return pl.pallas_call(
        _init_body,
        grid=_grid,
        in_specs=[_row_spec(64), _full_spec((64, _H))],
        out_specs=(_row_spec(_HH), _row_spec(_HH)),
        out_shape=_half_shapes,
    )(x_pad, w1p)


def _tc_stage(pa, pb, aggs, w2, w1n, vecs):
    return pl.pallas_call(
        _stage_body,
        grid=_grid,
        in_specs=[_row_spec(_HH), _row_spec(_HH), _agg_spec,
                  _full_spec((_H, _H)), _full_spec((_H, _H)),
                  _full_spec((4, _H))],
        out_specs=(_row_spec(_HH), _row_spec(_HH)),
        out_shape=_half_shapes,
    )(pa, pb, aggs, w2, w1n, vecs)


def _tc_final(pa, pb, aggs, w2, vecs, f1w, f1b, f2w, f2b):
    return pl.pallas_call(
        _final_body,
        grid=_grid,
        in_specs=[_row_spec(_HH), _row_spec(_HH), _agg_spec,
                  _full_spec((_H, _H)), _full_spec((4, _H)),
                  _full_spec((_H, _H)), _full_spec((1, _H)),
                  _full_spec((_H, 2)), _full_spec((1, 2))],
        out_specs=_row_spec(2),
        out_shape=jax.ShapeDtypeStruct((_N, 2), jnp.float32),
    )(pa, pb, aggs, w2, vecs, f1w, f1b, f2w, f2b)


def kernel(x, edge_index, dropout, params):
    del dropout

    mlps = params["mlps"]
    inv_std = 1.0 / jnp.sqrt(1.0 + 1e-5)

    src = edge_index[0]
    dst = edge_index[1]
    npad = _E_PAD - _E
    ar = jnp.arange(npad, dtype=jnp.int32)
    pad_src = (ar * 97) % _N
    pad_dst = _N + ar % (_N_PAD - _N)
    src_r = jnp.concatenate([src, pad_src]).reshape(_NW, _NCHUNK, _K)
    dst_r = jnp.concatenate([dst, pad_dst]).reshape(_NW, _NCHUNK, _K)

    x_pad = jnp.pad(x, ((0, 0), (0, 64 - _F_IN)))
    w1p = jnp.pad(mlps[0]["W1"], ((0, 64 - _F_IN), (0, 0)))

    pa, pb = _tc_init(x_pad, w1p)
    for i in range(5):
        aggs = _seg_sum_sc(pa, pb, src_r, dst_r)
        g = params["bng"][i] * inv_std
        vecs = jnp.stack([mlps[i]["b1"], mlps[i]["b2"], g, params["bnb"][i]])
        if i < 4:
            pa, pb = _tc_stage(pa, pb, aggs, mlps[i]["W2"],
                               mlps[i + 1]["W1"], vecs)
        else:
            out = _tc_final(pa, pb, aggs, mlps[i]["W2"], vecs,
                            params["fc1W"], params["fc1b"].reshape(1, _H),
                            params["fc2W"], params["fc2b"].reshape(1, 2))
    return out


# speedup vs baseline: 10.5403x; 1.8409x over previous
import functools

import jax
import jax.numpy as jnp
from jax import lax
from jax.experimental import pallas as pl
from jax.experimental.pallas import tpu as pltpu
from jax.experimental.pallas import tpu_sc as plsc

_N = 50000
_E = 800000
_F_IN = 61
_H = 32
_HH = 16

_NW = 32
_K = 128
_NCHUNK = 196
_E_PAD = _NW * _K * _NCHUNK
_N_PAD = 50176
_RPT = _N_PAD // 16
_ZR = 224
_NBUF = 4

_BLK = 5000


_sc_mesh = plsc.VectorSubcoreMesh(core_axis_name="c", subcore_axis_name="s")


@functools.partial(
    pl.kernel,
    mesh=_sc_mesh,
    out_type=jax.ShapeDtypeStruct((2, 2, _N_PAD, _HH), jnp.float32),
    scratch_types=[
        pltpu.VMEM((_NCHUNK, _K), jnp.int32),
        pltpu.VMEM((_NCHUNK, _K), jnp.int32),
        pltpu.VMEM((_NBUF, _K, _HH), jnp.float32),
        pltpu.VMEM((_ZR, _HH), jnp.float32),
        pltpu.VMEM((_ZR, _HH), jnp.float32),
        pltpu.VMEM_SHARED((_N_PAD, _HH), jnp.float32),
        pltpu.SemaphoreType.DMA,
        pltpu.SemaphoreType.DMA,
        pltpu.SemaphoreType.DMA,
        pltpu.SemaphoreType.DMA,
    ],
    compiler_params=pltpu.CompilerParams(use_tc_tiling_on_sc=False),
)
def _seg_sum_sc(pa_hbm, pb_hbm, src_hbm, dst_hbm, out_hbm,
                src_v, dst_v, rows_v, zbuf, obuf, agg,
                sem0, sem1, sem2, sem3):
    sems = (sem0, sem1, sem2, sem3)
    cid = lax.axis_index("c")
    sid = lax.axis_index("s")
    wid = sid * 2 + cid

    pltpu.sync_copy(src_hbm.at[wid], src_v)
    pltpu.sync_copy(dst_hbm.at[wid], dst_v)

    z16 = jnp.zeros((16,), jnp.float32)

    def _zrow(i, carry):
        zbuf[i, pl.ds(0, _HH)] = z16
        return carry

    lax.fori_loop(0, _ZR, _zrow, 0)

    row0 = sid * _RPT

    for ph, p_hbm in enumerate((pa_hbm, pb_hbm)):
        def _zcp(k, carry):
            pltpu.sync_copy(zbuf, agg.at[pl.ds(row0 + k * _ZR, _ZR)])
            return carry

        lax.fori_loop(0, _RPT // _ZR, _zcp, 0)
        plsc.subcore_barrier()

        def _issue(j, b):
            pltpu.async_copy(p_hbm.at[src_v.at[j]], rows_v.at[b], sems[b])

        def _wait(j, b):
            pltpu.make_async_copy(
                p_hbm.at[src_v.at[j]], rows_v.at[b], sems[b]).wait()

        def _scat(j, b):
            pltpu.sync_copy(rows_v.at[b], agg.at[dst_v.at[j]], add=True)

        for b in range(_NBUF):
            _issue(b, b)

        def _grp(jj, carry):
            j0 = jj * _NBUF
            for b in range(_NBUF):
                _wait(j0 + b, b)
                _scat(j0 + b, b)
                _issue(j0 + b + _NBUF, b)
            return carry

        lax.fori_loop(0, _NCHUNK // _NBUF - 1, _grp, 0)
        jt = _NCHUNK - _NBUF
        for b in range(_NBUF):
            _wait(jt + b, b)
            _scat(jt + b, b)
        plsc.subcore_barrier()

        def _ocp(k, carry):
            sl = pl.ds(row0 + k * _ZR, _ZR)
            pltpu.sync_copy(agg.at[sl], obuf)
            pltpu.sync_copy(obuf, out_hbm.at[ph, cid, sl])
            return carry

        lax.fori_loop(0, _RPT // _ZR, _ocp, 0)
        plsc.subcore_barrier()


def _split_store(res, oa_ref, ob_ref):
    oa_ref[...] = res[:, :_HH]
    ob_ref[...] = res[:, _HH:]


def _init_body(x_ref, w_ref, oa_ref, ob_ref):
    res = jnp.dot(x_ref[...], w_ref[...], preferred_element_type=jnp.float32)
    _split_store(res, oa_ref, ob_ref)


def _gin_epilogue(pa_ref, pb_ref, a_ref, w2_ref, v_ref):
    qa = pa_ref[...] + a_ref[0, 0] + a_ref[0, 1] + v_ref[0:1, :_HH]
    qb = pb_ref[...] + a_ref[1, 0] + a_ref[1, 1] + v_ref[0:1, _HH:]
    r = jnp.maximum(jnp.concatenate([qa, qb], axis=1), 0.0)
    z = jnp.dot(r, w2_ref[...], preferred_element_type=jnp.float32)
    u = jnp.maximum(z + v_ref[1:2, :], 0.0)
    return u * v_ref[2:3, :] + v_ref[3:4, :]


def _stage_body(pa_ref, pb_ref, a_ref, w2_ref, w1n_ref, v_ref,
                oa_ref, ob_ref):
    h = _gin_epilogue(pa_ref, pb_ref, a_ref, w2_ref, v_ref)
    res = jnp.dot(h, w1n_ref[...], preferred_element_type=jnp.float32)
    _split_store(res, oa_ref, ob_ref)


def _final_body(pa_ref, pb_ref, a_ref, w2_ref, v_ref, f1w_ref, f1b_ref,
                f2w_ref, f2b_ref, o_ref):
    h = _gin_epilogue(pa_ref, pb_ref, a_ref, w2_ref, v_ref)
    t = jnp.maximum(
        jnp.dot(h, f1w_ref[...], preferred_element_type=jnp.float32)
        + f1b_ref[0:1, :], 0.0)
    o = jnp.dot(t, f2w_ref[...], preferred_element_type=jnp.float32) \
        + f2b_ref[0:1, :]
    m = jnp.max(o, axis=1, keepdims=True)
    lse = m + jnp.log(jnp.sum(jnp.exp(o - m), axis=1, keepdims=True))
    o_ref[...] = o - lse


def _row_spec(cols):
    return pl.BlockSpec((_BLK, cols), lambda b: (b, 0))


def _full_spec(shape):
    nd = len(shape)
    return pl.BlockSpec(shape, lambda b: (0,) * nd)


_agg_spec = pl.BlockSpec((2, 2, _BLK, _HH), lambda b: (0, 0, b, 0))
_grid = (_N // _BLK,)
_half_shapes = (jax.ShapeDtypeStruct((_N, _HH), jnp.float32),
                jax.ShapeDtypeStruct((_N, _HH), jnp.float32))


def _tc_init(x_pad, w1p):
    return pl.pallas_call(
        _init_body,
        grid=_grid,
        in_specs=[_row_spec(64), _full_spec((64, _H))],
        out_specs=(_row_spec(_HH), _row_spec(_HH)),
        out_shape=_half_shapes,
    )(x_pad, w1p)


def _tc_stage(pa, pb, aggs, w2, w1n, vecs):
    return pl.pallas_call(
        _stage_body,
        grid=_grid,
        in_specs=[_row_spec(_HH), _row_spec(_HH), _agg_spec,
                  _full_spec((_H, _H)), _full_spec((_H, _H)),
                  _full_spec((4, _H))],
        out_specs=(_row_spec(_HH), _row_spec(_HH)),
        out_shape=_half_shapes,
    )(pa, pb, aggs, w2, w1n, vecs)


def _tc_final(pa, pb, aggs, w2, vecs, f1w, f1b, f2w, f2b):
    return pl.pallas_call(
        _final_body,
        grid=_grid,
        in_specs=[_row_spec(_HH), _row_spec(_HH), _agg_spec,
                  _full_spec((_H, _H)), _full_spec((4, _H)),
                  _full_spec((_H, _H)), _full_spec((1, _H)),
                  _full_spec((_H, 2)), _full_spec((1, 2))],
        out_specs=_row_spec(2),
        out_shape=jax.ShapeDtypeStruct((_N, 2), jnp.float32),
    )(pa, pb, aggs, w2, vecs, f1w, f1b, f2w, f2b)


def kernel(x, edge_index, dropout, params):
    del dropout

    mlps = params["mlps"]
    inv_std = 1.0 / jnp.sqrt(1.0 + 1e-5)

    src = edge_index[0]
    dst = edge_index[1]
    npad = _E_PAD - _E
    ar = jnp.arange(npad, dtype=jnp.int32)
    pad_src = (ar * 97) % _N
    pad_dst = _N + ar % (_N_PAD - _N)
    src_r = jnp.concatenate([src, pad_src]).reshape(_NW, _NCHUNK, _K)
    dst_r = jnp.concatenate([dst, pad_dst]).reshape(_NW, _NCHUNK, _K)

    x_pad = jnp.pad(x, ((0, 0), (0, 64 - _F_IN)))
    w1p = jnp.pad(mlps[0]["W1"], ((0, 64 - _F_IN), (0, 0)))

    pa, pb = _tc_init(x_pad, w1p)
    for i in range(5):
        aggs = _seg_sum_sc(pa, pb, src_r, dst_r)
        g = params["bng"][i] * inv_std
        vecs = jnp.stack([mlps[i]["b1"], mlps[i]["b2"], g, params["bnb"][i]])
        if i < 4:
            pa, pb = _tc_stage(pa, pb, aggs, mlps[i]["W2"],
                               mlps[i + 1]["W1"], vecs)
        else:
            out = _tc_final(pa, pb, aggs, mlps[i]["W2"], vecs,
                            params["fc1W"], params["fc1b"].reshape(1, _H),
                            params["fc2W"], params["fc2b"].reshape(1, 2))
    return out


# speedup vs baseline: 15.8795x; 1.5066x over previous
import functools

import jax
import jax.numpy as jnp
from jax import lax
from jax.experimental import pallas as pl
from jax.experimental.pallas import tpu as pltpu
from jax.experimental.pallas import tpu_sc as plsc

_N = 50000
_E = 800000
_H = 32
_HH = 16

_NW = 32
_K = 128
_NCHUNK = 196
_E_PAD = _NW * _K * _NCHUNK
_ER = _E_PAD // _K
_N_PAD = 50176
_RPT = _N_PAD // 16
_ZR = 224
_NBUF = 4

_NPK = 51200
_RP = _NPK // 8
_RA = _N_PAD // 8

_BLK = 5120
_BPK = _BLK // 8


def _seg_sum_body(pa_hbm, pb_hbm, e_hbm, out_hbm,
                  src_v, dst_v, rows_v, zbuf, obuf, agg,
                  sem0, sem1, sem2, sem3):
    sems = (sem0, sem1, sem2, sem3)
    cid = lax.axis_index("c")
    sid = lax.axis_index("s")
    wid = sid * 2 + cid

    pltpu.sync_copy(e_hbm.at[0, pl.ds(wid * _NCHUNK, _NCHUNK)], src_v)
    pltpu.sync_copy(e_hbm.at[1, pl.ds(wid * _NCHUNK, _NCHUNK)], dst_v)

    z16 = jnp.zeros((16,), jnp.float32)

    def _zrow(i, carry):
        zbuf[i, pl.ds(0, _HH)] = z16
        return carry

    lax.fori_loop(0, _ZR, _zrow, 0)

    row0 = sid * _RPT

    for ph, p_hbm in enumerate((pa_hbm, pb_hbm)):
        def _zcp(k, carry):
            pltpu.sync_copy(zbuf, agg.at[pl.ds(row0 + k * _ZR, _ZR)])
            return carry

        lax.fori_loop(0, _RPT // _ZR, _zcp, 0)
        plsc.subcore_barrier()

        def _issue(j, b):
            pltpu.async_copy(p_hbm.at[src_v.at[j]], rows_v.at[b], sems[b])

        def _wait(j, b):
            pltpu.make_async_copy(
                p_hbm.at[src_v.at[j]], rows_v.at[b], sems[b]).wait()

        def _scat(j, b):
            pltpu.sync_copy(rows_v.at[b], agg.at[dst_v.at[j]], add=True)

        for b in range(_NBUF):
            _issue(b, b)

        def _grp(jj, carry):
            j0 = jj * _NBUF
            for b in range(_NBUF):
                _wait(j0 + b, b)
                _scat(j0 + b, b)
                _issue(j0 + b + _NBUF, b)
            return carry

        lax.fori_loop(0, _NCHUNK // _NBUF - 1, _grp, 0)
        jt = _NCHUNK - _NBUF
        for b in range(_NBUF):
            _wait(jt + b, b)
            _scat(jt + b, b)
        plsc.subcore_barrier()

        def _ocp(k, carry):
            sl = pl.ds(row0 + k * _ZR, _ZR)
            pltpu.sync_copy(agg.at[sl], obuf)
            pltpu.sync_copy(obuf, out_hbm.at[ph, cid, sl])
            return carry

        lax.fori_loop(0, _RPT // _ZR, _ocp, 0)
        plsc.subcore_barrier()


@functools.cache
def _make_seg_sum_sc():
    mesh = plsc.VectorSubcoreMesh(core_axis_name="c", subcore_axis_name="s")
    return pl.kernel(
        _seg_sum_body,
        mesh=mesh,
        out_type=jax.ShapeDtypeStruct((2, 2, _N_PAD, _HH), jnp.float32),
        scratch_types=[
            pltpu.VMEM((_NCHUNK, _K), jnp.int32),
            pltpu.VMEM((_NCHUNK, _K), jnp.int32),
            pltpu.VMEM((_NBUF, _K, _HH), jnp.float32),
            pltpu.VMEM((_ZR, _HH), jnp.float32),
            pltpu.VMEM((_ZR, _HH), jnp.float32),
            pltpu.VMEM_SHARED((_N_PAD, _HH), jnp.float32),
            pltpu.SemaphoreType.DMA,
            pltpu.SemaphoreType.DMA,
            pltpu.SemaphoreType.DMA,
            pltpu.SemaphoreType.DMA,
        ],
        compiler_params=pltpu.CompilerParams(use_tc_tiling_on_sc=False),
    )


def _seg_sum_sc(pa_flat, pb_flat, e_r):
    return _make_seg_sum_sc()(pa_flat, pb_flat, e_r)


def _init_body(x_ref, w_ref, o_ref):
    o_ref[...] = jnp.dot(x_ref[...], w_ref[...],
                         preferred_element_type=jnp.float32)


def _packed_gin(pa_ref, pb_ref, ag_ref, w2_ref, v_ref):
    qa = pa_ref[...] + ag_ref[0, 0] + ag_ref[0, 1] + v_ref[0, 0]
    qb = pb_ref[...] + ag_ref[1, 0] + ag_ref[1, 1] + v_ref[0, 1]
    ra = jnp.maximum(qa, 0.0)
    rb = jnp.maximum(qb, 0.0)
    za = jnp.dot(ra, w2_ref[0], preferred_element_type=jnp.float32) \
        + jnp.dot(rb, w2_ref[1], preferred_element_type=jnp.float32)
    zb = jnp.dot(ra, w2_ref[2], preferred_element_type=jnp.float32) \
        + jnp.dot(rb, w2_ref[3], preferred_element_type=jnp.float32)
    ua = jnp.maximum(za + v_ref[1, 0], 0.0)
    ub = jnp.maximum(zb + v_ref[1, 1], 0.0)
    ha = ua * v_ref[2, 0] + v_ref[3, 0]
    hb = ub * v_ref[2, 1] + v_ref[3, 1]
    return ha, hb


def _stage_body(pa_ref, pb_ref, ag_ref, w2_ref, w1n_ref, v_ref,
                oa_ref, ob_ref):
    ha, hb = _packed_gin(pa_ref, pb_ref, ag_ref, w2_ref, v_ref)
    oa_ref[...] = jnp.dot(ha, w1n_ref[0], preferred_element_type=jnp.float32) \
        + jnp.dot(hb, w1n_ref[1], preferred_element_type=jnp.float32)
    ob_ref[...] = jnp.dot(ha, w1n_ref[2], preferred_element_type=jnp.float32) \
        + jnp.dot(hb, w1n_ref[3], preferred_element_type=jnp.float32)


def _final_body(pa_ref, pb_ref, ag_ref, w2_ref, v_ref, f1bd_ref, f1vt_ref,
                kab_ref, sp_ref, f2t_ref, o_ref):
    ha, hb = _packed_gin(pa_ref, pb_ref, ag_ref, w2_ref, v_ref)
    ta = jnp.maximum(
        jnp.dot(ha, f1bd_ref[0], preferred_element_type=jnp.float32)
        + jnp.dot(hb, f1bd_ref[1], preferred_element_type=jnp.float32)
        + f1vt_ref[0], 0.0)
    tb = jnp.maximum(
        jnp.dot(ha, f1bd_ref[2], preferred_element_type=jnp.float32)
        + jnp.dot(hb, f1bd_ref[3], preferred_element_type=jnp.float32)
        + f1vt_ref[1], 0.0)
    o = jnp.dot(ta, kab_ref[0], preferred_element_type=jnp.float32) \
        + jnp.dot(tb, kab_ref[1], preferred_element_type=jnp.float32) \
        + f2t_ref[0:1, :]
    m = jnp.maximum(o, jnp.dot(o, sp_ref[1],
                               preferred_element_type=jnp.float32))
    e = jnp.exp(o - m)
    se = jnp.dot(e, sp_ref[0], preferred_element_type=jnp.float32)
    o_ref[...] = o - (m + jnp.log(se))


def _pk_spec():
    return pl.BlockSpec((_BPK, 128), lambda b: (b, 0))


def _full_spec(shape):
    nd = len(shape)
    return pl.BlockSpec(shape, lambda b: (0,) * nd)


_agg_spec = pl.BlockSpec((2, 2, _BPK, 128), lambda b: (0, 0, b, 0))
_grid = (_NPK // _BLK,)
_pk_shapes = (jax.ShapeDtypeStruct((_RP, 128), jnp.float32),
              jax.ShapeDtypeStruct((_RP, 128), jnp.float32))


def _tc_init(x, w1):
    return pl.pallas_call(
        _init_body,
        grid=_grid,
        in_specs=[pl.BlockSpec((_BLK, 61), lambda b: (b, 0)),
                  _full_spec((61, _H))],
        out_specs=pl.BlockSpec((_BLK, _H), lambda b: (b, 0)),
        out_shape=jax.ShapeDtypeStruct((_N, _H), jnp.float32),
    )(x, w1)


def _tc_stage(pa, pb, aggp, w2bd, w1nbd, vt):
    return pl.pallas_call(
        _stage_body,
        grid=_grid,
        in_specs=[_pk_spec(), _pk_spec(), _agg_spec,
                  _full_spec((4, 128, 128)), _full_spec((4, 128, 128)),
                  _full_spec((4, 2, 128))],
        out_specs=(_pk_spec(), _pk_spec()),
        out_shape=_pk_shapes,
    )(pa, pb, aggp, w2bd, w1nbd, vt)


def _tc_final(pa, pb, aggp, w2bd, vt, f1bd, f1vt, kab, sp, f2t):
    return pl.pallas_call(
        _final_body,
        grid=_grid,
        in_specs=[_pk_spec(), _pk_spec(), _agg_spec,
                  _full_spec((4, 128, 128)), _full_spec((4, 2, 128)),
                  _full_spec((4, 128, 128)), _full_spec((2, 128)),
                  _full_spec((2, 128, 16)), _full_spec((2, 16, 16)),
                  _full_spec((1, 16))],
        out_specs=pl.BlockSpec((_BPK, 16), lambda b: (b, 0)),
        out_shape=jax.ShapeDtypeStruct((_RP, 16), jnp.float32),
    )(pa, pb, aggp, w2bd, vt, f1bd, f1vt, kab, sp, f2t)


def _bd4(w):
    eye8 = jnp.eye(8, dtype=jnp.float32)
    blocks = [w[:_HH, :_HH], w[_HH:, :_HH], w[:_HH, _HH:], w[_HH:, _HH:]]
    return jnp.stack([jnp.kron(eye8, blk) for blk in blocks])


def _vtile(vs):
    v = jnp.stack(vs)
    return jnp.tile(v.reshape(4, 2, _HH), (1, 1, 8))


def kernel(x, edge_index, dropout, params):
    del dropout

    mlps = params["mlps"]
    inv_std = 1.0 / jnp.sqrt(1.0 + 1e-5)

    npad = _E_PAD - _E
    pad_blk = jnp.concatenate(
        [jnp.zeros((1, npad), jnp.int32),
         jnp.full((1, npad), _N, jnp.int32)])
    e_r = jnp.concatenate([edge_index, pad_blk], axis=1).reshape(2, _ER, _K)

    p0 = _tc_init(x, mlps[0]["W1"])
    zrows = ((0, _NPK - _N), (0, 0))
    pa = jnp.pad(p0[:, :_HH], zrows).reshape(_RP, 128)
    pb = jnp.pad(p0[:, _HH:], zrows).reshape(_RP, 128)

    eye8 = jnp.eye(8, dtype=jnp.float32)
    for i in range(5):
        aggs = _seg_sum_sc(pa.reshape(_NPK, _HH), pb.reshape(_NPK, _HH), e_r)
        aggp = aggs.reshape(2, 2, _RA, 128)
        g = params["bng"][i] * inv_std
        w2bd = _bd4(mlps[i]["W2"])
        vt = _vtile([mlps[i]["b1"], mlps[i]["b2"], g, params["bnb"][i]])
        if i < 4:
            pa, pb = _tc_stage(pa, pb, aggp, w2bd, _bd4(mlps[i + 1]["W1"]),
                               vt)
        else:
            f2w = params["fc2W"]
            kab = jnp.stack([jnp.kron(eye8, f2w[:_HH, :]),
                             jnp.kron(eye8, f2w[_HH:, :])])
            sp = jnp.stack([
                jnp.kron(eye8, jnp.ones((2, 2), jnp.float32)),
                jnp.kron(eye8, jnp.array([[0., 1.], [1., 0.]], jnp.float32)),
            ])
            out_pk = _tc_final(
                pa, pb, aggp, w2bd, vt,
                _bd4(params["fc1W"]),
                jnp.tile(params["fc1b"].reshape(2, _HH), (1, 8)),
                kab, sp,
                jnp.tile(params["fc2b"], 8).reshape(1, 16))
    return out_pk.reshape(_NPK, 2)[:_N]


# speedup vs baseline: 17.7872x; 1.1201x over previous
import functools

import jax
import jax.numpy as jnp
from jax import lax
from jax.experimental import pallas as pl
from jax.experimental.pallas import tpu as pltpu
from jax.experimental.pallas import tpu_sc as plsc

_N = 50000
_E = 800000
_H = 32
_HH = 16

_NW = 32
_K = 128
_NCHUNK = 200
_E_PAD = _NW * _K * _NCHUNK
_ER = _E_PAD // _K
_N_PAD = 50176
_RPT = _N_PAD // 16
_ZR = 224
_NBUF = 4

_NPK = 51200
_RP = _NPK // 8
_RA = _N_PAD // 8

_BLK = 5120
_BPK = _BLK // 8


def _seg_sum_body(pa_hbm, pb_hbm, e_hbm, out_hbm,
                  src_v, dst_v, rows_v, zbuf, obuf, agg,
                  sem0, sem1, sem2, sem3):
    sems = (sem0, sem1, sem2, sem3)
    cid = lax.axis_index("c")
    sid = lax.axis_index("s")
    wid = sid * 2 + cid

    pltpu.sync_copy(e_hbm.at[0, wid], src_v)
    pltpu.sync_copy(e_hbm.at[1, wid], dst_v)

    z16 = jnp.zeros((16,), jnp.float32)

    def _zrow(i, carry):
        zbuf[i, pl.ds(0, _HH)] = z16
        return carry

    lax.fori_loop(0, _ZR, _zrow, 0)

    row0 = sid * _RPT

    for ph, p_hbm in enumerate((pa_hbm, pb_hbm)):
        def _zcp(k, carry):
            pltpu.sync_copy(zbuf, agg.at[pl.ds(row0 + k * _ZR, _ZR)])
            return carry

        lax.fori_loop(0, _RPT // _ZR, _zcp, 0)
        plsc.subcore_barrier()

        def _issue(j, b):
            pltpu.async_copy(p_hbm.at[src_v.at[j]], rows_v.at[b], sems[b])

        def _wait(j, b):
            pltpu.make_async_copy(
                p_hbm.at[src_v.at[j]], rows_v.at[b], sems[b]).wait()

        def _scat(j, b):
            pltpu.sync_copy(rows_v.at[b], agg.at[dst_v.at[j]], add=True)

        for b in range(_NBUF):
            _issue(b, b)

        def _grp(jj, carry):
            j0 = jj * _NBUF
            for b in range(_NBUF):
                _wait(j0 + b, b)
                _scat(j0 + b, b)
                _issue(j0 + b + _NBUF, b)
            return carry

        lax.fori_loop(0, _NCHUNK // _NBUF - 1, _grp, 0)
        jt = _NCHUNK - _NBUF
        for b in range(_NBUF):
            _wait(jt + b, b)
            _scat(jt + b, b)
        plsc.subcore_barrier()

        def _ocp(k, carry):
            sl = pl.ds(row0 + k * _ZR, _ZR)
            pltpu.sync_copy(agg.at[sl], obuf)
            pltpu.sync_copy(obuf, out_hbm.at[ph, cid, sl])
            return carry

        lax.fori_loop(0, _RPT // _ZR, _ocp, 0)
        plsc.subcore_barrier()


@functools.cache
def _make_seg_sum_sc():
    mesh = plsc.VectorSubcoreMesh(core_axis_name="c", subcore_axis_name="s")
    return pl.kernel(
        _seg_sum_body,
        mesh=mesh,
        out_type=jax.ShapeDtypeStruct((2, 2, _N_PAD, _HH), jnp.float32),
        scratch_types=[
            pltpu.VMEM((_NCHUNK, _K), jnp.int32),
            pltpu.VMEM((_NCHUNK, _K), jnp.int32),
            pltpu.VMEM((_NBUF, _K, _HH), jnp.float32),
            pltpu.VMEM((_ZR, _HH), jnp.float32),
            pltpu.VMEM((_ZR, _HH), jnp.float32),
            pltpu.VMEM_SHARED((_N_PAD, _HH), jnp.float32),
            pltpu.SemaphoreType.DMA,
            pltpu.SemaphoreType.DMA,
            pltpu.SemaphoreType.DMA,
            pltpu.SemaphoreType.DMA,
        ],
        compiler_params=pltpu.CompilerParams(use_tc_tiling_on_sc=False),
    )


def _seg_sum_sc(pa_flat, pb_flat, e_r):
    return _make_seg_sum_sc()(pa_flat, pb_flat, e_r)


def _init_body(x_ref, w_ref, o_ref):
    o_ref[...] = jnp.dot(x_ref[...], w_ref[...],
                         preferred_element_type=jnp.float32)


def _packed_gin(pa_ref, pb_ref, ag_ref, w2_ref, v_ref):
    qa = pa_ref[...] + ag_ref[0, 0] + ag_ref[0, 1] + v_ref[0, 0]
    qb = pb_ref[...] + ag_ref[1, 0] + ag_ref[1, 1] + v_ref[0, 1]
    ra = jnp.maximum(qa, 0.0)
    rb = jnp.maximum(qb, 0.0)
    za = jnp.dot(ra, w2_ref[0], preferred_element_type=jnp.float32) \
        + jnp.dot(rb, w2_ref[1], preferred_element_type=jnp.float32)
    zb = jnp.dot(ra, w2_ref[2], preferred_element_type=jnp.float32) \
        + jnp.dot(rb, w2_ref[3], preferred_element_type=jnp.float32)
    ua = jnp.maximum(za + v_ref[1, 0], 0.0)
    ub = jnp.maximum(zb + v_ref[1, 1], 0.0)
    ha = ua * v_ref[2, 0] + v_ref[3, 0]
    hb = ub * v_ref[2, 1] + v_ref[3, 1]
    return ha, hb


def _stage_body(pa_ref, pb_ref, ag_ref, w2_ref, w1n_ref, v_ref,
                oa_ref, ob_ref):
    ha, hb = _packed_gin(pa_ref, pb_ref, ag_ref, w2_ref, v_ref)
    oa_ref[...] = jnp.dot(ha, w1n_ref[0], preferred_element_type=jnp.float32) \
        + jnp.dot(hb, w1n_ref[1], preferred_element_type=jnp.float32)
    ob_ref[...] = jnp.dot(ha, w1n_ref[2], preferred_element_type=jnp.float32) \
        + jnp.dot(hb, w1n_ref[3], preferred_element_type=jnp.float32)


def _final_body(pa_ref, pb_ref, ag_ref, w2_ref, v_ref, f1bd_ref, f1vt_ref,
                kab_ref, sp_ref, f2t_ref, o_ref):
    ha, hb = _packed_gin(pa_ref, pb_ref, ag_ref, w2_ref, v_ref)
    ta = jnp.maximum(
        jnp.dot(ha, f1bd_ref[0], preferred_element_type=jnp.float32)
        + jnp.dot(hb, f1bd_ref[1], preferred_element_type=jnp.float32)
        + f1vt_ref[0], 0.0)
    tb = jnp.maximum(
        jnp.dot(ha, f1bd_ref[2], preferred_element_type=jnp.float32)
        + jnp.dot(hb, f1bd_ref[3], preferred_element_type=jnp.float32)
        + f1vt_ref[1], 0.0)
    o = jnp.dot(ta, kab_ref[0], preferred_element_type=jnp.float32) \
        + jnp.dot(tb, kab_ref[1], preferred_element_type=jnp.float32) \
        + f2t_ref[0:1, :]
    m = jnp.maximum(o, jnp.dot(o, sp_ref[1],
                               preferred_element_type=jnp.float32))
    e = jnp.exp(o - m)
    se = jnp.dot(e, sp_ref[0], preferred_element_type=jnp.float32)
    o_ref[...] = o - (m + jnp.log(se))


def _pk_spec():
    return pl.BlockSpec((_BPK, 128), lambda b: (b, 0))


def _full_spec(shape):
    nd = len(shape)
    return pl.BlockSpec(shape, lambda b: (0,) * nd)


_agg_spec = pl.BlockSpec((2, 2, _BPK, 128), lambda b: (0, 0, b, 0))
_grid = (_NPK // _BLK,)
_pk_shapes = (jax.ShapeDtypeStruct((_RP, 128), jnp.float32),
              jax.ShapeDtypeStruct((_RP, 128), jnp.float32))


def _tc_init(x, w1):
    return pl.pallas_call(
        _init_body,
        grid=_grid,
        in_specs=[pl.BlockSpec((_BLK, 61), lambda b: (b, 0)),
                  _full_spec((61, _H))],
        out_specs=pl.BlockSpec((_BLK, _H), lambda b: (b, 0)),
        out_shape=jax.ShapeDtypeStruct((_N, _H), jnp.float32),
    )(x, w1)


def _tc_stage(pa, pb, aggp, w2bd, w1nbd, vt):
    return pl.pallas_call(
        _stage_body,
        grid=_grid,
        in_specs=[_pk_spec(), _pk_spec(), _agg_spec,
                  _full_spec((4, 128, 128)), _full_spec((4, 128, 128)),
                  _full_spec((4, 2, 128))],
        out_specs=(_pk_spec(), _pk_spec()),
        out_shape=_pk_shapes,
    )(pa, pb, aggp, w2bd, w1nbd, vt)


def _tc_final(pa, pb, aggp, w2bd, vt, f1bd, f1vt, kab, sp, f2t):
    return pl.pallas_call(
        _final_body,
        grid=_grid,
        in_specs=[_pk_spec(), _pk_spec(), _agg_spec,
                  _full_spec((4, 128, 128)), _full_spec((4, 2, 128)),
                  _full_spec((4, 128, 128)), _full_spec((2, 128)),
                  _full_spec((2, 128, 16)), _full_spec((2, 16, 16)),
                  _full_spec((1, 16))],
        out_specs=pl.BlockSpec((_BPK, 16), lambda b: (b, 0)),
        out_shape=jax.ShapeDtypeStruct((_RP, 16), jnp.float32),
    )(pa, pb, aggp, w2bd, vt, f1bd, f1vt, kab, sp, f2t)


def _bd4(w):
    eye8 = jnp.eye(8, dtype=jnp.float32)
    blocks = [w[:_HH, :_HH], w[_HH:, :_HH], w[:_HH, _HH:], w[_HH:, _HH:]]
    return jnp.stack([jnp.kron(eye8, blk) for blk in blocks])


def _vtile(vs):
    v = jnp.stack(vs)
    return jnp.tile(v.reshape(4, 2, _HH), (1, 1, 8))


def kernel(x, edge_index, dropout, params):
    del dropout

    mlps = params["mlps"]
    inv_std = 1.0 / jnp.sqrt(1.0 + 1e-5)

    npad = _E_PAD - _E
    ar = jnp.arange(npad, dtype=jnp.int32)
    pad_blk = jnp.stack([ar % _N, _N + ar % (_N_PAD - _N)])
    e_r = jnp.concatenate([edge_index, pad_blk], axis=1).reshape(
        2, _NW, _NCHUNK, _K)

    p0 = _tc_init(x, mlps[0]["W1"])
    zrows = ((0, _NPK - _N), (0, 0))
    pa = jnp.pad(p0[:, :_HH], zrows).reshape(_RP, 128)
    pb = jnp.pad(p0[:, _HH:], zrows).reshape(_RP, 128)

    eye8 = jnp.eye(8, dtype=jnp.float32)
    for i in range(5):
        aggs = _seg_sum_sc(pa.reshape(_NPK, _HH), pb.reshape(_NPK, _HH), e_r)
        aggp = aggs.reshape(2, 2, _RA, 128)
        g = params["bng"][i] * inv_std
        w2bd = _bd4(mlps[i]["W2"])
        vt = _vtile([mlps[i]["b1"], mlps[i]["b2"], g, params["bnb"][i]])
        if i < 4:
            pa, pb = _tc_stage(pa, pb, aggp, w2bd, _bd4(mlps[i + 1]["W1"]),
                               vt)
        else:
            f2w = params["fc2W"]
            kab = jnp.stack([jnp.kron(eye8, f2w[:_HH, :]),
                             jnp.kron(eye8, f2w[_HH:, :])])
            sp = jnp.stack([
                jnp.kron(eye8, jnp.ones((2, 2), jnp.float32)),
                jnp.kron(eye8, jnp.array([[0., 1.], [1., 0.]], jnp.float32)),
            ])
            out_pk = _tc_final(
                pa, pb, aggp, w2bd, vt,
                _bd4(params["fc1W"]),
                jnp.tile(params["fc1b"].reshape(2, _HH), (1, 8)),
                kab, sp,
                jnp.tile(params["fc2b"], 8).reshape(1, 16))
    return out_pk.reshape(_NPK, 2)[:_N]


# speedup vs baseline: 20.6072x; 1.1585x over previous
import functools

import jax
import jax.numpy as jnp
from jax import lax
from jax.experimental import pallas as pl
from jax.experimental.pallas import tpu as pltpu
from jax.experimental.pallas import tpu_sc as plsc

_N = 50000
_E = 800000
_H = 32
_HH = 16

_NW = 32
_K = 128
_NCHUNK = 200
_E_PAD = _NW * _K * _NCHUNK
_ER = _E_PAD // _K
_N_PAD = 50176
_RPT = _N_PAD // 16
_ZR = 224
_NBUF = 8

_NPK = 51200
_RP = _NPK // 8
_RA = _N_PAD // 8

_BLK = 5120
_BPK = _BLK // 8


def _seg_sum_body(pa_hbm, pb_hbm, e_hbm, out_hbm,
                  src_v, dst_v, rows_v, zbuf, obuf, agg, *sems):
    gs = sems[:_NBUF]
    ss = sems[_NBUF:2 * _NBUF]
    zs = sems[2 * _NBUF]
    cid = lax.axis_index("c")
    sid = lax.axis_index("s")
    wid = sid * 2 + cid

    pltpu.sync_copy(e_hbm.at[0, wid], src_v)
    pltpu.sync_copy(e_hbm.at[1, wid], dst_v)

    z16 = jnp.zeros((16,), jnp.float32)

    def _zrow(i, carry):
        zbuf[i, pl.ds(0, _HH)] = z16
        return carry

    lax.fori_loop(0, _ZR, _zrow, 0)

    row0 = sid * _RPT
    nz = _RPT // _ZR

    for ph, p_hbm in enumerate((pa_hbm, pb_hbm)):
        def _zcp(k, carry):
            pltpu.async_copy(zbuf, agg.at[pl.ds(row0 + k * _ZR, _ZR)], zs)
            return carry

        lax.fori_loop(0, nz, _zcp, 0)

        def _zdr(k, carry):
            pltpu.make_async_copy(zbuf, agg.at[pl.ds(row0, _ZR)], zs).wait()
            return carry

        lax.fori_loop(0, nz, _zdr, 0)
        plsc.subcore_barrier()

        def _gissue(j, b):
            pltpu.async_copy(p_hbm.at[src_v.at[j]], rows_v.at[b], gs[b])

        def _gwait(j, b):
            pltpu.make_async_copy(
                p_hbm.at[src_v.at[j]], rows_v.at[b], gs[b]).wait()

        def _sissue(j, b):
            pltpu.async_copy(rows_v.at[b], agg.at[dst_v.at[j]], ss[b],
                             add=True)

        def _swait(j, b):
            pltpu.make_async_copy(
                rows_v.at[b], agg.at[dst_v.at[j]], ss[b]).wait()

        for b in range(_NBUF):
            _gissue(b, b)

        def _grp(jj, carry):
            j0 = jj * _NBUF
            for b in range(_NBUF):
                _gwait(j0 + b, b)
                _sissue(j0 + b, b)
            for b in range(_NBUF):
                _swait(j0 + b, b)
                _gissue(j0 + b + _NBUF, b)
            return carry

        lax.fori_loop(0, _NCHUNK // _NBUF - 1, _grp, 0)
        jt = _NCHUNK - _NBUF
        for b in range(_NBUF):
            _gwait(jt + b, b)
            _sissue(jt + b, b)
        for b in range(_NBUF):
            _swait(jt + b, b)
        plsc.subcore_barrier()

        def _ocp(k, carry):
            sl = pl.ds(row0 + k * _ZR, _ZR)
            pltpu.sync_copy(agg.at[sl], obuf)
            pltpu.sync_copy(obuf, out_hbm.at[ph, cid, sl])
            return carry

        lax.fori_loop(0, nz, _ocp, 0)
        plsc.subcore_barrier()


@functools.cache
def _make_seg_sum_sc():
    mesh = plsc.VectorSubcoreMesh(core_axis_name="c", subcore_axis_name="s")
    return pl.kernel(
        _seg_sum_body,
        mesh=mesh,
        out_type=jax.ShapeDtypeStruct((2, 2, _N_PAD, _HH), jnp.float32),
        scratch_types=[
            pltpu.VMEM((_NCHUNK, _K), jnp.int32),
            pltpu.VMEM((_NCHUNK, _K), jnp.int32),
            pltpu.VMEM((_NBUF, _K, _HH), jnp.float32),
            pltpu.VMEM((_ZR, _HH), jnp.float32),
            pltpu.VMEM((_ZR, _HH), jnp.float32),
            pltpu.VMEM_SHARED((_N_PAD, _HH), jnp.float32),
        ] + [pltpu.SemaphoreType.DMA] * (2 * _NBUF + 1),
        compiler_params=pltpu.CompilerParams(use_tc_tiling_on_sc=False),
    )


def _seg_sum_sc(pa_flat, pb_flat, e_r):
    return _make_seg_sum_sc()(pa_flat, pb_flat, e_r)


def _init_body(x_ref, w_ref, o_ref):
    o_ref[...] = jnp.dot(x_ref[...], w_ref[...],
                         preferred_element_type=jnp.float32)


def _packed_gin(pa_ref, pb_ref, ag_ref, w2_ref, v_ref):
    qa = pa_ref[...] + ag_ref[0, 0] + ag_ref[0, 1] + v_ref[0, 0]
    qb = pb_ref[...] + ag_ref[1, 0] + ag_ref[1, 1] + v_ref[0, 1]
    ra = jnp.maximum(qa, 0.0)
    rb = jnp.maximum(qb, 0.0)
    za = jnp.dot(ra, w2_ref[0], preferred_element_type=jnp.float32) \
        + jnp.dot(rb, w2_ref[1], preferred_element_type=jnp.float32)
    zb = jnp.dot(ra, w2_ref[2], preferred_element_type=jnp.float32) \
        + jnp.dot(rb, w2_ref[3], preferred_element_type=jnp.float32)
    ua = jnp.maximum(za + v_ref[1, 0], 0.0)
    ub = jnp.maximum(zb + v_ref[1, 1], 0.0)
    ha = ua * v_ref[2, 0] + v_ref[3, 0]
    hb = ub * v_ref[2, 1] + v_ref[3, 1]
    return ha, hb


def _stage_body(pa_ref, pb_ref, ag_ref, w2_ref, w1n_ref, v_ref,
                oa_ref, ob_ref):
    ha, hb = _packed_gin(pa_ref, pb_ref, ag_ref, w2_ref, v_ref)
    oa_ref[...] = jnp.dot(ha, w1n_ref[0], preferred_element_type=jnp.float32) \
        + jnp.dot(hb, w1n_ref[1], preferred_element_type=jnp.float32)
    ob_ref[...] = jnp.dot(ha, w1n_ref[2], preferred_element_type=jnp.float32) \
        + jnp.dot(hb, w1n_ref[3], preferred_element_type=jnp.float32)


def _final_body(pa_ref, pb_ref, ag_ref, w2_ref, v_ref, f1bd_ref, f1vt_ref,
                kab_ref, sp_ref, f2t_ref, o_ref):
    ha, hb = _packed_gin(pa_ref, pb_ref, ag_ref, w2_ref, v_ref)
    ta = jnp.maximum(
        jnp.dot(ha, f1bd_ref[0], preferred_element_type=jnp.float32)
        + jnp.dot(hb, f1bd_ref[1], preferred_element_type=jnp.float32)
        + f1vt_ref[0], 0.0)
    tb = jnp.maximum(
        jnp.dot(ha, f1bd_ref[2], preferred_element_type=jnp.float32)
        + jnp.dot(hb, f1bd_ref[3], preferred_element_type=jnp.float32)
        + f1vt_ref[1], 0.0)
    o = jnp.dot(ta, kab_ref[0], preferred_element_type=jnp.float32) \
        + jnp.dot(tb, kab_ref[1], preferred_element_type=jnp.float32) \
        + f2t_ref[0:1, :]
    m = jnp.maximum(o, jnp.dot(o, sp_ref[1],
                               preferred_element_type=jnp.float32))
    e = jnp.exp(o - m)
    se = jnp.dot(e, sp_ref[0], preferred_element_type=jnp.float32)
    o_ref[...] = o - (m + jnp.log(se))


def _pk_spec():
    return pl.BlockSpec((_BPK, 128), lambda b: (b, 0))


def _full_spec(shape):
    nd = len(shape)
    return pl.BlockSpec(shape, lambda b: (0,) * nd)


_agg_spec = pl.BlockSpec((2, 2, _BPK, 128), lambda b: (0, 0, b, 0))
_grid = (_NPK // _BLK,)
_pk_shapes = (jax.ShapeDtypeStruct((_RP, 128), jnp.float32),
              jax.ShapeDtypeStruct((_RP, 128), jnp.float32))


def _tc_init(x, w1):
    return pl.pallas_call(
        _init_body,
        grid=_grid,
        in_specs=[pl.BlockSpec((_BLK, 61), lambda b: (b, 0)),
                  _full_spec((61, _H))],
        out_specs=pl.BlockSpec((_BLK, _H), lambda b: (b, 0)),
        out_shape=jax.ShapeDtypeStruct((_N, _H), jnp.float32),
    )(x, w1)


def _tc_stage(pa, pb, aggp, w2bd, w1nbd, vt):
    return pl.pallas_call(
        _stage_body,
        grid=_grid,
        in_specs=[_pk_spec(), _pk_spec(), _agg_spec,
                  _full_spec((4, 128, 128)), _full_spec((4, 128, 128)),
                  _full_spec((4, 2, 128))],
        out_specs=(_pk_spec(), _pk_spec()),
        out_shape=_pk_shapes,
    )(pa, pb, aggp, w2bd, w1nbd, vt)


def _tc_final(pa, pb, aggp, w2bd, vt, f1bd, f1vt, kab, sp, f2t):
    return pl.pallas_call(
        _final_body,
        grid=_grid,
        in_specs=[_pk_spec(), _pk_spec(), _agg_spec,
                  _full_spec((4, 128, 128)), _full_spec((4, 2, 128)),
                  _full_spec((4, 128, 128)), _full_spec((2, 128)),
                  _full_spec((2, 128, 16)), _full_spec((2, 16, 16)),
                  _full_spec((1, 16))],
        out_specs=pl.BlockSpec((_BPK, 16), lambda b: (b, 0)),
        out_shape=jax.ShapeDtypeStruct((_RP, 16), jnp.float32),
    )(pa, pb, aggp, w2bd, vt, f1bd, f1vt, kab, sp, f2t)


def _bd4(w):
    eye8 = jnp.eye(8, dtype=jnp.float32)
    blocks = [w[:_HH, :_HH], w[_HH:, :_HH], w[:_HH, _HH:], w[_HH:, _HH:]]
    return jnp.stack([jnp.kron(eye8, blk) for blk in blocks])


def _vtile(vs):
    v = jnp.stack(vs)
    return jnp.tile(v.reshape(4, 2, _HH), (1, 1, 8))


def kernel(x, edge_index, dropout, params):
    del dropout

    mlps = params["mlps"]
    inv_std = 1.0 / jnp.sqrt(1.0 + 1e-5)

    npad = _E_PAD - _E
    ar = jnp.arange(npad, dtype=jnp.int32)
    pad_blk = jnp.stack([ar % _N, _N + ar % (_N_PAD - _N)])
    e_r = jnp.concatenate([edge_index, pad_blk], axis=1).reshape(
        2, _NW, _NCHUNK, _K)

    p0 = _tc_init(x, mlps[0]["W1"])
    zrows = ((0, _NPK - _N), (0, 0))
    pa = jnp.pad(p0[:, :_HH], zrows).reshape(_RP, 128)
    pb = jnp.pad(p0[:, _HH:], zrows).reshape(_RP, 128)

    eye8 = jnp.eye(8, dtype=jnp.float32)
    for i in range(5):
        aggs = _seg_sum_sc(pa.reshape(_NPK, _HH), pb.reshape(_NPK, _HH), e_r)
        aggp = aggs.reshape(2, 2, _RA, 128)
        g = params["bng"][i] * inv_std
        w2bd = _bd4(mlps[i]["W2"])
        vt = _vtile([mlps[i]["b1"], mlps[i]["b2"], g, params["bnb"][i]])
        if i < 4:
            pa, pb = _tc_stage(pa, pb, aggp, w2bd, _bd4(mlps[i + 1]["W1"]),
                               vt)
        else:
            f2w = params["fc2W"]
            kab = jnp.stack([jnp.kron(eye8, f2w[:_HH, :]),
                             jnp.kron(eye8, f2w[_HH:, :])])
            sp = jnp.stack([
                jnp.kron(eye8, jnp.ones((2, 2), jnp.float32)),
                jnp.kron(eye8, jnp.array([[0., 1.], [1., 0.]], jnp.float32)),
            ])
            out_pk = _tc_final(
                pa, pb, aggp, w2bd, vt,
                _bd4(params["fc1W"]),
                jnp.tile(params["fc1b"].reshape(2, _HH), (1, 8)),
                kab, sp,
                jnp.tile(params["fc2b"], 8).reshape(1, 16))
    return out_pk.reshape(_NPK, 2)[:_N]


# speedup vs baseline: 21.3701x; 1.0370x over previous
import functools

import jax
import jax.numpy as jnp
from jax import lax
from jax.experimental import pallas as pl
from jax.experimental.pallas import tpu as pltpu
from jax.experimental.pallas import tpu_sc as plsc

_N = 50000
_E = 800000
_H = 32
_HH = 16

_NW = 32
_K = 256
_NCHUNK = 104
_E_PAD = _NW * _K * _NCHUNK
_N_PAD = 50176
_RPT = _N_PAD // 16
_ZR = 224
_NBUF = 4

_NPK = 51200
_RP = _NPK // 8
_RA = _N_PAD // 8

_BLK = 5120
_BPK = _BLK // 8


def _seg_sum_body(pa_hbm, pb_hbm, e_hbm, out_hbm,
                  src_v, dst_v, rows_v, zbuf, obuf, agg, *sems):
    gs = sems[:_NBUF]
    zs = sems[_NBUF]
    cid = lax.axis_index("c")
    sid = lax.axis_index("s")
    wid = sid * 2 + cid

    pltpu.sync_copy(e_hbm.at[0, wid], src_v)
    pltpu.sync_copy(e_hbm.at[1, wid], dst_v)

    z16 = jnp.zeros((16,), jnp.float32)

    def _zrow(i, carry):
        zbuf[i, pl.ds(0, _HH)] = z16
        return carry

    lax.fori_loop(0, _ZR, _zrow, 0)

    row0 = sid * _RPT
    nz = _RPT // _ZR

    for ph, p_hbm in enumerate((pa_hbm, pb_hbm)):
        def _zcp(k, carry):
            pltpu.async_copy(zbuf, agg.at[pl.ds(row0 + k * _ZR, _ZR)], zs)
            return carry

        lax.fori_loop(0, nz, _zcp, 0)

        def _zdr(k, carry):
            pltpu.make_async_copy(zbuf, agg.at[pl.ds(row0, _ZR)], zs).wait()
            return carry

        lax.fori_loop(0, nz, _zdr, 0)
        plsc.subcore_barrier()

        def _gissue(j, b):
            pltpu.async_copy(p_hbm.at[src_v.at[j]], rows_v.at[b], gs[b])

        def _gwait(j, b):
            pltpu.make_async_copy(
                p_hbm.at[src_v.at[j]], rows_v.at[b], gs[b]).wait()

        def _scat(j, b):
            pltpu.sync_copy(rows_v.at[b], agg.at[dst_v.at[j]], add=True)

        for b in range(_NBUF):
            _gissue(b, b)

        def _grp(jj, carry):
            j0 = jj * _NBUF
            for b in range(_NBUF):
                _gwait(j0 + b, b)
                _scat(j0 + b, b)
                _gissue(j0 + b + _NBUF, b)
            return carry

        lax.fori_loop(0, _NCHUNK // _NBUF - 1, _grp, 0)
        jt = _NCHUNK - _NBUF
        for b in range(_NBUF):
            _gwait(jt + b, b)
            _scat(jt + b, b)
        plsc.subcore_barrier()

        def _ocp(k, carry):
            sl = pl.ds(row0 + k * _ZR, _ZR)
            pltpu.sync_copy(agg.at[sl], obuf)
            pltpu.sync_copy(obuf, out_hbm.at[ph, cid, sl])
            return carry

        lax.fori_loop(0, nz, _ocp, 0)
        plsc.subcore_barrier()


@functools.cache
def _make_seg_sum_sc():
    mesh = plsc.VectorSubcoreMesh(core_axis_name="c", subcore_axis_name="s")
    return pl.kernel(
        _seg_sum_body,
        mesh=mesh,
        out_type=jax.ShapeDtypeStruct((2, 2, _N_PAD, _HH), jnp.float32),
        scratch_types=[
            pltpu.VMEM((_NCHUNK, _K), jnp.int32),
            pltpu.VMEM((_NCHUNK, _K), jnp.int32),
            pltpu.VMEM((_NBUF, _K, _HH), jnp.float32),
            pltpu.VMEM((_ZR, _HH), jnp.float32),
            pltpu.VMEM((_ZR, _HH), jnp.float32),
            pltpu.VMEM_SHARED((_N_PAD, _HH), jnp.float32),
        ] + [pltpu.SemaphoreType.DMA] * (_NBUF + 1),
        compiler_params=pltpu.CompilerParams(use_tc_tiling_on_sc=False),
    )


def _seg_sum_sc(pa_flat, pb_flat, e_r):
    return _make_seg_sum_sc()(pa_flat, pb_flat, e_r)


def _init_body(x_ref, w_ref, o_ref):
    o_ref[...] = jnp.dot(x_ref[...], w_ref[...],
                         preferred_element_type=jnp.float32)


def _packed_gin(pa_ref, pb_ref, ag_ref, w2_ref, v_ref):
    qa = pa_ref[...] + ag_ref[0, 0] + ag_ref[0, 1] + v_ref[0, 0]
    qb = pb_ref[...] + ag_ref[1, 0] + ag_ref[1, 1] + v_ref[0, 1]
    ra = jnp.maximum(qa, 0.0)
    rb = jnp.maximum(qb, 0.0)
    za = jnp.dot(ra, w2_ref[0], preferred_element_type=jnp.float32) \
        + jnp.dot(rb, w2_ref[1], preferred_element_type=jnp.float32)
    zb = jnp.dot(ra, w2_ref[2], preferred_element_type=jnp.float32) \
        + jnp.dot(rb, w2_ref[3], preferred_element_type=jnp.float32)
    ua = jnp.maximum(za + v_ref[1, 0], 0.0)
    ub = jnp.maximum(zb + v_ref[1, 1], 0.0)
    ha = ua * v_ref[2, 0] + v_ref[3, 0]
    hb = ub * v_ref[2, 1] + v_ref[3, 1]
    return ha, hb


def _stage_body(pa_ref, pb_ref, ag_ref, w2_ref, w1n_ref, v_ref,
                oa_ref, ob_ref):
    ha, hb = _packed_gin(pa_ref, pb_ref, ag_ref, w2_ref, v_ref)
    oa_ref[...] = jnp.dot(ha, w1n_ref[0], preferred_element_type=jnp.float32) \
        + jnp.dot(hb, w1n_ref[1], preferred_element_type=jnp.float32)
    ob_ref[...] = jnp.dot(ha, w1n_ref[2], preferred_element_type=jnp.float32) \
        + jnp.dot(hb, w1n_ref[3], preferred_element_type=jnp.float32)


def _final_body(pa_ref, pb_ref, ag_ref, w2_ref, v_ref, f1bd_ref, f1vt_ref,
                kab_ref, sp_ref, f2t_ref, o_ref):
    ha, hb = _packed_gin(pa_ref, pb_ref, ag_ref, w2_ref, v_ref)
    ta = jnp.maximum(
        jnp.dot(ha, f1bd_ref[0], preferred_element_type=jnp.float32)
        + jnp.dot(hb, f1bd_ref[1], preferred_element_type=jnp.float32)
        + f1vt_ref[0], 0.0)
    tb = jnp.maximum(
        jnp.dot(ha, f1bd_ref[2], preferred_element_type=jnp.float32)
        + jnp.dot(hb, f1bd_ref[3], preferred_element_type=jnp.float32)
        + f1vt_ref[1], 0.0)
    o = jnp.dot(ta, kab_ref[0], preferred_element_type=jnp.float32) \
        + jnp.dot(tb, kab_ref[1], preferred_element_type=jnp.float32) \
        + f2t_ref[0:1, :]
    m = jnp.maximum(o, jnp.dot(o, sp_ref[1],
                               preferred_element_type=jnp.float32))
    e = jnp.exp(o - m)
    se = jnp.dot(e, sp_ref[0], preferred_element_type=jnp.float32)
    o_ref[...] = o - (m + jnp.log(se))


def _pk_spec():
    return pl.BlockSpec((_BPK, 128), lambda b: (b, 0))


def _full_spec(shape):
    nd = len(shape)
    return pl.BlockSpec(shape, lambda b: (0,) * nd)


_agg_spec = pl.BlockSpec((2, 2, _BPK, 128), lambda b: (0, 0, b, 0))
_grid = (_NPK // _BLK,)
_pk_shapes = (jax.ShapeDtypeStruct((_RP, 128), jnp.float32),
              jax.ShapeDtypeStruct((_RP, 128), jnp.float32))


def _tc_init(x, w1):
    return pl.pallas_call(
        _init_body,
        grid=_grid,
        in_specs=[pl.BlockSpec((_BLK, 61), lambda b: (b, 0)),
                  _full_spec((61, _H))],
        out_specs=pl.BlockSpec((_BLK, _H), lambda b: (b, 0)),
        out_shape=jax.ShapeDtypeStruct((_N, _H), jnp.float32),
    )(x, w1)


def _tc_stage(pa, pb, aggp, w2bd, w1nbd, vt):
    return pl.pallas_call(
        _stage_body,
        grid=_grid,
        in_specs=[_pk_spec(), _pk_spec(), _agg_spec,
                  _full_spec((4, 128, 128)), _full_spec((4, 128, 128)),
                  _full_spec((4, 2, 128))],
        out_specs=(_pk_spec(), _pk_spec()),
        out_shape=_pk_shapes,
    )(pa, pb, aggp, w2bd, w1nbd, vt)


def _tc_final(pa, pb, aggp, w2bd, vt, f1bd, f1vt, kab, sp, f2t):
    return pl.pallas_call(
        _final_body,
        grid=_grid,
        in_specs=[_pk_spec(), _pk_spec(), _agg_spec,
                  _full_spec((4, 128, 128)), _full_spec((4, 2, 128)),
                  _full_spec((4, 128, 128)), _full_spec((2, 128)),
                  _full_spec((2, 128, 16)), _full_spec((2, 16, 16)),
                  _full_spec((1, 16))],
        out_specs=pl.BlockSpec((_BPK, 16), lambda b: (b, 0)),
        out_shape=jax.ShapeDtypeStruct((_RP, 16), jnp.float32),
    )(pa, pb, aggp, w2bd, vt, f1bd, f1vt, kab, sp, f2t)


def _bd4(w):
    eye8 = jnp.eye(8, dtype=jnp.float32)
    blocks = [w[:_HH, :_HH], w[_HH:, :_HH], w[:_HH, _HH:], w[_HH:, _HH:]]
    return jnp.stack([jnp.kron(eye8, blk) for blk in blocks])


def _vtile(vs):
    v = jnp.stack(vs)
    return jnp.tile(v.reshape(4, 2, _HH), (1, 1, 8))


def kernel(x, edge_index, dropout, params):
    del dropout

    mlps = params["mlps"]
    inv_std = 1.0 / jnp.sqrt(1.0 + 1e-5)

    npad = _E_PAD - _E
    ar = jnp.arange(npad, dtype=jnp.int32)
    pad_blk = jnp.stack([ar % _N, _N + ar % (_N_PAD - _N)])
    e_r = jnp.concatenate([edge_index, pad_blk], axis=1).reshape(
        2, _NW, _NCHUNK, _K)

    p0 = _tc_init(x, mlps[0]["W1"])
    zrows = ((0, _NPK - _N), (0, 0))
    pa = jnp.pad(p0[:, :_HH], zrows).reshape(_RP, 128)
    pb = jnp.pad(p0[:, _HH:], zrows).reshape(_RP, 128)

    eye8 = jnp.eye(8, dtype=jnp.float32)
    for i in range(5):
        aggs = _seg_sum_sc(pa.reshape(_NPK, _HH), pb.reshape(_NPK, _HH), e_r)
        aggp = aggs.reshape(2, 2, _RA, 128)
        g = params["bng"][i] * inv_std
        w2bd = _bd4(mlps[i]["W2"])
        vt = _vtile([mlps[i]["b1"], mlps[i]["b2"], g, params["bnb"][i]])
        if i < 4:
            pa, pb = _tc_stage(pa, pb, aggp, w2bd, _bd4(mlps[i + 1]["W1"]),
                               vt)
        else:
            f2w = params["fc2W"]
            kab = jnp.stack([jnp.kron(eye8, f2w[:_HH, :]),
                             jnp.kron(eye8, f2w[_HH:, :])])
            sp = jnp.stack([
                jnp.kron(eye8, jnp.ones((2, 2), jnp.float32)),
                jnp.kron(eye8, jnp.array([[0., 1.], [1., 0.]], jnp.float32)),
            ])
            out_pk = _tc_final(
                pa, pb, aggp, w2bd, vt,
                _bd4(params["fc1W"]),
                jnp.tile(params["fc1b"].reshape(2, _HH), (1, 8)),
                kab, sp,
                jnp.tile(params["fc2b"], 8).reshape(1, 16))
    return out_pk.reshape(_NPK, 2)[:_N]
